# baseline (speedup 1.0000x reference)
import functools

import jax
import jax.numpy as jnp
from jax.experimental import pallas as pl
from jax.experimental.pallas import tpu as pltpu

_KEEP_RATIO = 0.7
_ZSCORE_EPS = 1e-3
_EPS = 1e-6


def _stats_gate_kernel(x_ref, rm_ref, fm_ref, rs_ref, fs_ref, gate_ref,
                       m_ref, ia_ref, *, cb, bb, c, hw, b, kth):
    cstep = pl.program_id(0)
    bstep = pl.program_id(1)
    ncs = pl.num_programs(0)
    nbs = pl.num_programs(1)

    xb = x_ref[...]
    m = jnp.sum(xb, axis=(2, 3)) * (1.0 / hw)
    m_ref[pl.ds(pl.multiple_of(bstep * bb, bb), bb), :] = m

    @pl.when(bstep == nbs - 1)
    def _imp():
        csl = (slice(None), pl.ds(pl.multiple_of(cstep * cb, cb), cb))
        rm = rm_ref[csl]
        fm = fm_ref[csl]
        rs = rs_ref[csl]
        fs = fs_ref[csl]
        mm = m_ref[...]
        zr = jnp.abs((mm - rm) / (rs + _ZSCORE_EPS))
        zf = jnp.abs((mm - fm) / (fs + _ZSCORE_EPS))
        imp = jnp.abs(fm - rm) / (jnp.minimum(zr, zf) + _EPS)
        ia_blk = jnp.sum(imp, axis=0, keepdims=True) * (1.0 / b)
        ia_ref[csl] = ia_blk

    @pl.when((cstep == ncs - 1) & (bstep == nbs - 1))
    def _gate():
        v = ia_ref[...]
        vi = jax.lax.bitcast_convert_type(v, jnp.int32)
        hi0 = jnp.max(vi)
        lo0 = jnp.int32(-1)

        def body(_, carry):
            lo, hi = carry
            mid = lo + (hi - lo) // 2
            midf = jax.lax.bitcast_convert_type(mid, jnp.float32)
            cntm = jnp.sum((v <= midf).astype(jnp.int32))
            pred = cntm >= (kth + 1)
            return (jnp.where(pred, lo, mid), jnp.where(pred, mid, hi))

        _, hi = jax.lax.fori_loop(0, 32, body, (lo0, hi0))
        thr = jax.lax.bitcast_convert_type(hi, jnp.float32)
        gate_ref[...] = (v >= thr).astype(jnp.float32)


def _mul_kernel(gate_ref, x_ref, out_ref, *, cb):
    c0 = pl.program_id(0) * cb
    for i in range(cb):
        g = gate_ref[0, c0 + i]
        out_ref[:, i] = x_ref[:, i] * g


def kernel(x, real_mean, fake_mean, real_std, fake_std):
    B, C, H, W = x.shape
    HW = H * W
    kth = max(0, min(int((1.0 - _KEEP_RATIO) * C), C - 1))

    rm = real_mean.reshape(1, C)
    fm = fake_mean.reshape(1, C)
    rs = real_std.reshape(1, C)
    fs = fake_std.reshape(1, C)

    cb1 = 128
    bb1 = 8
    stats_fn = functools.partial(_stats_gate_kernel, cb=cb1, bb=bb1, c=C,
                                 hw=HW, b=B, kth=kth)
    gate = pl.pallas_call(
        stats_fn,
        grid=(C // cb1, B // bb1),
        in_specs=[
            pl.BlockSpec((bb1, cb1, H, W), lambda i, j: (j, i, 0, 0)),
            pl.BlockSpec((1, C), lambda i, j: (0, 0)),
            pl.BlockSpec((1, C), lambda i, j: (0, 0)),
            pl.BlockSpec((1, C), lambda i, j: (0, 0)),
            pl.BlockSpec((1, C), lambda i, j: (0, 0)),
        ],
        out_shape=jax.ShapeDtypeStruct((1, C), jnp.float32),
        out_specs=pl.BlockSpec((1, C), lambda i, j: (0, 0)),
        scratch_shapes=[pltpu.VMEM((B, cb1), jnp.float32),
                        pltpu.VMEM((1, C), jnp.float32)],
    )(x, rm, fm, rs, fs)

    cb2 = 32
    mul_fn = functools.partial(_mul_kernel, cb=cb2)
    out = pl.pallas_call(
        mul_fn,
        grid=(C // cb2,),
        in_specs=[
            pl.BlockSpec(memory_space=pltpu.SMEM),
            pl.BlockSpec((B, cb2, H, W), lambda i: (0, i, 0, 0)),
        ],
        out_shape=jax.ShapeDtypeStruct((B, C, H, W), jnp.float32),
        out_specs=pl.BlockSpec((B, cb2, H, W), lambda i: (0, i, 0, 0)),
    )(gate, x)

    return out, gate.reshape(C)


# speedup vs baseline: 9.8207x; 9.8207x over previous
import functools

import jax
import jax.numpy as jnp
from jax.experimental import pallas as pl
from jax.experimental.pallas import tpu as pltpu

_KEEP_RATIO = 0.7
_ZSCORE_EPS = 1e-3
_EPS = 1e-6


def _stats_gate_kernel(x_ref, rm_ref, fm_ref, rs_ref, fs_ref, gate_ref,
                       m_ref, *, bb, c, hw, b, kth):
    step = pl.program_id(0)
    nsteps = pl.num_programs(0)

    xb = x_ref[...]
    m = jnp.sum(xb, axis=(1, 2)) * (1.0 / hw)
    m_ref[pl.ds(pl.multiple_of(step * bb, bb), bb), :] = m

    @pl.when(step == nsteps - 1)
    def _gate():
        rm = rm_ref[...]
        fm = fm_ref[...]
        rs = rs_ref[...]
        fs = fs_ref[...]
        mm = m_ref[...]
        zr = jnp.abs((mm - rm) / (rs + _ZSCORE_EPS))
        zf = jnp.abs((mm - fm) / (fs + _ZSCORE_EPS))
        imp = jnp.abs(fm - rm) / (jnp.minimum(zr, zf) + _EPS)
        v = jnp.sum(imp, axis=0, keepdims=True) * (1.0 / b)

        vi = jax.lax.bitcast_convert_type(v, jnp.int32)
        hi0 = jnp.max(vi)
        lo0 = jnp.int32(-1)

        def body(_, carry):
            lo, hi = carry
            mid = lo + (hi - lo) // 2
            midf = jax.lax.bitcast_convert_type(mid, jnp.float32)
            cntm = jnp.sum((v <= midf).astype(jnp.int32))
            pred = cntm >= (kth + 1)
            return (jnp.where(pred, lo, mid), jnp.where(pred, mid, hi))

        _, hi = jax.lax.fori_loop(0, 32, body, (lo0, hi0))
        thr = jax.lax.bitcast_convert_type(hi, jnp.float32)
        gate_ref[...] = (v >= thr).astype(jnp.float32)


def _mul_kernel(x_ref, gate_ref, out_ref):
    g = gate_ref[...]
    out_ref[...] = x_ref[...] * g[0]


def kernel(x, real_mean, fake_mean, real_std, fake_std):
    B, C, H, W = x.shape
    HW = H * W
    kth = max(0, min(int((1.0 - _KEEP_RATIO) * C), C - 1))

    xt = jnp.transpose(x, (0, 2, 3, 1))
    rm = real_mean.reshape(1, C)
    fm = fake_mean.reshape(1, C)
    rs = real_std.reshape(1, C)
    fs = fake_std.reshape(1, C)

    bb1 = 8
    stats_fn = functools.partial(_stats_gate_kernel, bb=bb1, c=C, hw=HW, b=B,
                                 kth=kth)
    gate = pl.pallas_call(
        stats_fn,
        grid=(B // bb1,),
        in_specs=[
            pl.BlockSpec((bb1, H, W, C), lambda i: (i, 0, 0, 0)),
            pl.BlockSpec((1, C), lambda i: (0, 0)),
            pl.BlockSpec((1, C), lambda i: (0, 0)),
            pl.BlockSpec((1, C), lambda i: (0, 0)),
            pl.BlockSpec((1, C), lambda i: (0, 0)),
        ],
        out_shape=jax.ShapeDtypeStruct((1, C), jnp.float32),
        out_specs=pl.BlockSpec((1, C), lambda i: (0, 0)),
        scratch_shapes=[pltpu.VMEM((B, C), jnp.float32)],
    )(xt, rm, fm, rs, fs)

    bb2 = 4
    outt = pl.pallas_call(
        _mul_kernel,
        grid=(B // bb2,),
        in_specs=[
            pl.BlockSpec((bb2, H, W, C), lambda i: (i, 0, 0, 0)),
            pl.BlockSpec((1, C), lambda i: (0, 0)),
        ],
        out_shape=jax.ShapeDtypeStruct((B, H, W, C), jnp.float32),
        out_specs=pl.BlockSpec((bb2, H, W, C), lambda i: (i, 0, 0, 0)),
    )(xt, gate)

    out = jnp.transpose(outt, (0, 3, 1, 2))
    return out, gate.reshape(C)


# speedup vs baseline: 11.2887x; 1.1495x over previous
import functools

import jax
import jax.numpy as jnp
from jax.experimental import pallas as pl
from jax.experimental.pallas import tpu as pltpu

_KEEP_RATIO = 0.7
_ZSCORE_EPS = 1e-3
_EPS = 1e-6


def _fused_kernel(x_ref, rm_ref, fm_ref, rs_ref, fs_ref, out_ref, gate_ref,
                  stash_ref, m_ref, g_ref, *, bb, c, hw, b, kth, n_stash):
    p = pl.program_id(0)
    i = pl.program_id(1)
    nsteps = pl.num_programs(1)

    @pl.when(p == 0)
    def _phase0():
        xb = x_ref[...]
        m_ref[i] = jnp.sum(xb, axis=(1, 2)) * (1.0 / hw)

        @pl.when(i < n_stash)
        def _stash():
            stash_ref[pl.ds(i * bb, bb)] = xb

        @pl.when(i == nsteps - 1)
        def _gate():
            rm = rm_ref[...]
            fm = fm_ref[...]
            rs = rs_ref[...]
            fs = fs_ref[...]
            mm = m_ref[...]
            zr = jnp.abs((mm - rm) / (rs + _ZSCORE_EPS))
            zf = jnp.abs((mm - fm) / (fs + _ZSCORE_EPS))
            imp = jnp.abs(fm - rm) / (jnp.minimum(zr, zf) + _EPS)
            v = (jnp.sum(imp, axis=(0, 1), keepdims=True) * (1.0 / b))[0]

            vi = jax.lax.bitcast_convert_type(v, jnp.int32)
            hi0 = jnp.max(vi)
            lo0 = jnp.int32(-1)

            def body(_, carry):
                lo, hi = carry
                mid = lo + (hi - lo) // 2
                midf = jax.lax.bitcast_convert_type(mid, jnp.float32)
                cntm = jnp.sum((v <= midf).astype(jnp.int32))
                pred = cntm >= (kth + 1)
                return (jnp.where(pred, lo, mid), jnp.where(pred, mid, hi))

            _, hi = jax.lax.fori_loop(0, 32, body, (lo0, hi0))
            thr = jax.lax.bitcast_convert_type(hi, jnp.float32)
            grow = (v >= thr).astype(jnp.float32)
            g_ref[...] = grow
            gate_ref[...] = grow

    @pl.when(p == 1)
    def _phase1():
        g = g_ref[...]

        @pl.when(i < n_stash)
        def _from_stash():
            out_ref[...] = stash_ref[pl.ds(i * bb, bb)] * g

        @pl.when(i >= n_stash)
        def _from_hbm():
            out_ref[...] = x_ref[...] * g


def kernel(x, real_mean, fake_mean, real_std, fake_std):
    B, C, H, W = x.shape
    HW = H * W
    kth = max(0, min(int((1.0 - _KEEP_RATIO) * C), C - 1))

    xt = jnp.transpose(x, (0, 2, 3, 1))
    rm = real_mean.reshape(1, C)
    fm = fake_mean.reshape(1, C)
    rs = real_std.reshape(1, C)
    fs = fake_std.reshape(1, C)

    bb = 2
    nsteps = B // bb
    n_stash = 12

    fused = functools.partial(_fused_kernel, bb=bb, c=C, hw=HW, b=B, kth=kth,
                              n_stash=n_stash)

    def x_idx(p, i):
        return (jnp.where((p == 1) & (i < n_stash), nsteps - 1, i), 0, 0, 0)

    outt, gate = pl.pallas_call(
        fused,
        grid=(2, nsteps),
        in_specs=[
            pl.BlockSpec((bb, H, W, C), x_idx),
            pl.BlockSpec((1, C), lambda p, i: (0, 0)),
            pl.BlockSpec((1, C), lambda p, i: (0, 0)),
            pl.BlockSpec((1, C), lambda p, i: (0, 0)),
            pl.BlockSpec((1, C), lambda p, i: (0, 0)),
        ],
        out_shape=(jax.ShapeDtypeStruct((B, H, W, C), jnp.float32),
                   jax.ShapeDtypeStruct((1, C), jnp.float32)),
        out_specs=(pl.BlockSpec((bb, H, W, C),
                                lambda p, i: (jnp.where(p == 0, 0, i), 0, 0, 0)),
                   pl.BlockSpec((1, C), lambda p, i: (0, 0))),
        scratch_shapes=[
            pltpu.VMEM((n_stash * bb, H, W, C), jnp.float32),
            pltpu.VMEM((nsteps, bb, C), jnp.float32),
            pltpu.VMEM((1, C), jnp.float32),
        ],
    )(xt, rm, fm, rs, fs)

    out = jnp.transpose(outt, (0, 3, 1, 2))
    return out, gate.reshape(C)
